# baseline (speedup 1.0000x reference)
import functools

import jax
import jax.numpy as jnp
from jax import lax
from jax.experimental import pallas as pl
from jax.experimental.pallas import tpu as pltpu
from jax.experimental.pallas import tpu_sc as plsc

LINPUT = 256
COUNT = 10
BUCKET_SIZE = 6
BATCH = 16384

NC = 2
NS = 16
NW = NC * NS
BPW = BATCH // NW
CHUNK = 128
NCHUNK = BPW // CHUNK
GROUPS = CHUNK // 16
XPAD = LINPUT + 1
KUNROLL = 2


def _make_sc_kernel():
    mesh = plsc.VectorSubcoreMesh(core_axis_name="c", subcore_axis_name="s")

    @functools.partial(
        pl.kernel,
        mesh=mesh,
        out_type=jax.ShapeDtypeStruct((BATCH,), jnp.float32),
        compiler_params=pltpu.CompilerParams(needs_layout_passes=False),
        scratch_types=[
            pltpu.VMEM(((LINPUT + 1) * 16,), jnp.float32),
            pltpu.VMEM((BPW,), jnp.int32),
            pltpu.VMEM((BPW,), jnp.float32),
            pltpu.VMEM((CHUNK, XPAD), jnp.float32),
            pltpu.VMEM((CHUNK, XPAD), jnp.float32),
            pltpu.SemaphoreType.DMA,
            pltpu.SemaphoreType.DMA,
        ],
    )
    def k(x_hbm, ply_hbm, wt_hbm, out_hbm,
          wt_v, ply_v, out_v, xb0, xb1, sem0, sem1):
        wid = lax.axis_index("s") * NC + lax.axis_index("c")
        base = wid * BPW

        xbufs = (xb0, xb1)
        sems = (sem0, sem1)

        cps = [None, None]
        cps[0] = pltpu.async_copy(
            x_hbm.at[pl.ds(base, CHUNK), :], xb0.at[:, pl.ds(0, LINPUT)], sem0)
        pltpu.sync_copy(wt_hbm, wt_v)
        pltpu.sync_copy(ply_hbm.at[pl.ds(base, BPW)], ply_v)

        iota16 = lax.iota(jnp.int32, 16)
        bias_v = wt_v[pl.ds(LINPUT * 16, 16)]

        for c in range(NCHUNK):
            cur = c % 2
            cps[cur].wait()
            if c + 1 < NCHUNK:
                nxt = (c + 1) % 2
                cps[nxt] = pltpu.async_copy(
                    x_hbm.at[pl.ds(base + (c + 1) * CHUNK, CHUNK), :],
                    xbufs[nxt].at[:, pl.ds(0, LINPUT)], sems[nxt])
            x_v = xbufs[cur]

            idxs = []
            rows = []
            inits = []
            for g in range(GROUPS):
                plyv = ply_v[pl.ds(c * CHUNK + g * 16, 16)]
                idxv = lax.div(plyv, jnp.int32(BUCKET_SIZE))
                idxs.append(idxv)
                rows.append(iota16 + (g * 16))
                inits.append(jnp.take_along_axis(
                    bias_v, idxv, axis=0, mode="promise_in_bounds"))

            @plsc.parallel_loop(0, LINPUT // KUNROLL, carry=tuple(inits),
                                unroll=8)
            def accs(kk, accs):
                accs = list(accs)
                for u in range(KUNROLL):
                    kf = kk * KUNROLL + u
                    wtk = wt_v[pl.ds(kf * 16, 16)]
                    colk = jnp.full((16,), 0, jnp.int32) + kf
                    for g in range(GROUPS):
                        wsel = jnp.take_along_axis(
                            wtk, idxs[g], axis=0, mode="promise_in_bounds")
                        xcol = plsc.load_gather(x_v, [rows[g], colk])
                        accs[g] = accs[g] + xcol * wsel
                return tuple(accs)
            for g in range(GROUPS):
                out_v[pl.ds(c * CHUNK + g * 16, 16)] = accs[g]

        pltpu.sync_copy(out_v, out_hbm.at[pl.ds(base, BPW)])

    return k


_sc_kernel = _make_sc_kernel()


@jax.jit
def kernel(x_pa, ply, W, b):
    wt = jnp.zeros((LINPUT + 1, 16), jnp.float32)
    wt = wt.at[:LINPUT, :COUNT].set(W.reshape(COUNT, LINPUT).T)
    wt = wt.at[LINPUT, :COUNT].set(b.reshape(COUNT))
    out = _sc_kernel(x_pa, ply, wt.reshape(-1))
    return out.reshape(BATCH, 1)


# speedup vs baseline: 1.6786x; 1.6786x over previous
import functools

import jax
import jax.numpy as jnp
from jax import lax
from jax.experimental import pallas as pl
from jax.experimental.pallas import tpu as pltpu
from jax.experimental.pallas import tpu_sc as plsc

LINPUT = 256
COUNT = 10
BUCKET_SIZE = 6
BATCH = 16384

NC = 2
NS = 16
NW = NC * NS
BPW = BATCH // NW
CHUNK = 128
NCHUNK = BPW // CHUNK
GROUPS = CHUNK // 16
MCH = LINPUT // 16


def _make_sc_kernel():
    mesh = plsc.VectorSubcoreMesh(core_axis_name="c", subcore_axis_name="s")

    @functools.partial(
        pl.kernel,
        mesh=mesh,
        out_type=jax.ShapeDtypeStruct((BATCH,), jnp.float32),
        compiler_params=pltpu.CompilerParams(needs_layout_passes=False),
        scratch_types=[
            pltpu.VMEM((COUNT * LINPUT + 16,), jnp.float32),
            pltpu.VMEM((BPW,), jnp.int32),
            pltpu.VMEM((BPW,), jnp.float32),
            pltpu.VMEM((CHUNK * LINPUT,), jnp.float32),
            pltpu.VMEM((CHUNK * LINPUT,), jnp.float32),
            pltpu.SemaphoreType.DMA,
            pltpu.SemaphoreType.DMA,
        ],
    )
    def k(x_hbm, ply_hbm, w_hbm, out_hbm,
          w_v, ply_v, out_v, xb0, xb1, sem0, sem1):
        wid = lax.axis_index("s") * NC + lax.axis_index("c")
        base = wid * BPW

        xbufs = (xb0, xb1)
        sems = (sem0, sem1)

        cps = [None, None]
        cps[0] = pltpu.async_copy(
            x_hbm.at[pl.ds(base * LINPUT, CHUNK * LINPUT)], xb0, sem0)
        pltpu.sync_copy(w_hbm, w_v)
        pltpu.sync_copy(ply_hbm.at[pl.ds(base, BPW)], ply_v)

        iota16 = lax.iota(jnp.int32, 16)
        lane_masks = [iota16 == j for j in range(16)]
        bias_v = w_v[pl.ds(COUNT * LINPUT, 16)]

        for c in range(NCHUNK):
            cur = c % 2
            cps[cur].wait()
            if c + 1 < NCHUNK:
                nxt = (c + 1) % 2
                cps[nxt] = pltpu.async_copy(
                    x_hbm.at[pl.ds((base + (c + 1) * CHUNK) * LINPUT,
                                   CHUNK * LINPUT)],
                    xbufs[nxt], sems[nxt])
            x_v = xbufs[cur]

            def gbody(g, _):
                plyv = ply_v[pl.ds(c * CHUNK + g * 16, 16)]
                idxv = lax.div(plyv, jnp.int32(BUCKET_SIZE))
                wbasev = idxv * LINPUT
                outvec = jnp.take_along_axis(
                    bias_v, idxv, axis=0, mode="promise_in_bounds")
                goff = g * (16 * LINPUT)
                for j in range(16):
                    wb = wbasev[j]
                    xoff = goff + j * LINPUT
                    acc = (x_v[pl.ds(xoff, 16)] * w_v[pl.ds(wb, 16)] +
                           x_v[pl.ds(xoff + 16, 16)] * w_v[pl.ds(wb + 16, 16)])
                    for m in range(2, MCH):
                        acc = acc + (x_v[pl.ds(xoff + m * 16, 16)] *
                                     w_v[pl.ds(wb + m * 16, 16)])
                    res = jnp.sum(acc)
                    outvec = jnp.where(lane_masks[j], res, outvec)
                out_v[pl.ds(c * CHUNK + g * 16, 16)] = outvec
                return 0

            lax.fori_loop(0, GROUPS, gbody, 0)

        pltpu.sync_copy(out_v, out_hbm.at[pl.ds(base, BPW)])

    return k


_sc_kernel = _make_sc_kernel()


@jax.jit
def kernel(x_pa, ply, W, b):
    x_flat = x_pa.reshape(BATCH * LINPUT)
    wb_flat = jnp.concatenate(
        [W.reshape(COUNT * LINPUT),
         jnp.pad(b.reshape(COUNT), (0, 16 - COUNT))])
    out = _sc_kernel(x_flat, ply, wb_flat)
    return out.reshape(BATCH, 1)
